# baseline (speedup 1.0000x reference)
import functools

import jax
import jax.numpy as jnp
from jax import lax
from jax.experimental import pallas as pl
from jax.experimental.pallas import tpu as pltpu
from jax.experimental.pallas import tpu_sc as plsc

N = 10000
E = 320000
T = 16
D = 128
H = 256
L = 256
C = 5

NTILES = 16
ROWS_PT = N // NTILES
K = 64
NCH = 320
NJ = NCH // 8
EPT = NCH * K
EPAD = NTILES * EPT
ACC_N = N + 16


@functools.lru_cache(maxsize=None)
def _make_seg_sum(G):
  mesh = plsc.VectorSubcoreMesh(core_axis_name="c", subcore_axis_name="s")

  @functools.partial(
      pl.kernel,
      out_type=jax.ShapeDtypeStruct((G, NTILES, ROWS_PT, D), jnp.float32),
      mesh=mesh,
      scratch_types=[
          [pltpu.VMEM((K,), jnp.int32) for _ in range(8)],
          [pltpu.VMEM((K,), jnp.int32) for _ in range(8)],
          [pltpu.VMEM((K, D), jnp.float32) for _ in range(4)],
          pltpu.VMEM_SHARED((ACC_N, D), jnp.float32),
          [pltpu.SemaphoreType.DMA for _ in range(8)],
          [pltpu.SemaphoreType.DMA for _ in range(4)],
          [pltpu.SemaphoreType.DMA for _ in range(4)],
      ],
  )
  def seg_sum(x_hbm, src_hbm, dst_hbm, zeros_hbm, out_hbm,
              srcv, dstv, rowsv, acc, isem, gsem, ssem):
    cid = lax.axis_index("c")
    sid = lax.axis_index("s")
    ebase = sid * EPT
    rbase = sid * ROWS_PT

    def idx_start(c, d):
      off = ebase + c * K
      pltpu.async_copy(src_hbm.at[pl.ds(off, K)], srcv[d], isem[d])
      pltpu.async_copy(dst_hbm.at[pl.ds(off, K)], dstv[d], isem[d])

    def idx_wait(d):
      pltpu.make_async_copy(src_hbm.at[pl.ds(0, K)], srcv[d], isem[d]).wait()
      pltpu.make_async_copy(dst_hbm.at[pl.ds(0, K)], dstv[d], isem[d]).wait()

    def gather_start(gg, r, d):
      pltpu.async_copy(x_hbm.at[gg].at[srcv[d]], rowsv[r], gsem[r])

    def gather_wait(gg, r, d):
      pltpu.make_async_copy(x_hbm.at[gg].at[srcv[d]], rowsv[r],
                            gsem[r]).wait()

    def scatter_start(r, d):
      pltpu.async_copy(rowsv[r], acc.at[dstv[d]], ssem[r], add=True)

    def scatter_wait(r, d):
      pltpu.make_async_copy(rowsv[r], acc.at[dstv[d]], ssem[r]).wait()

    def g_body(gl, carry):
      gg = 2 * gl + cid
      pltpu.sync_copy(zeros_hbm, acc.at[pl.ds(rbase, ROWS_PT)])
      plsc.subcore_barrier()

      for d in range(4):
        idx_start(d, d)
      idx_wait(0)
      gather_start(gg, 0, 0)
      idx_wait(1)
      gather_start(gg, 1, 1)

      def body(j, carry):
        for k in range(8):
          c = 8 * j + k
          r = k % 4
          if k >= 2:
            scatter_wait((k - 2) % 4, k - 2)
          else:
            @pl.when(j > 0)
            def _():
              scatter_wait((k - 2) % 4, (k - 2) % 8)
          gather_wait(gg, r, k)
          scatter_start(r, k)
          if k < 4:
            idx_start(c + 4, k + 4)
          else:
            @pl.when(j < NJ - 1)
            def _():
              idx_start(c + 4, (k + 4) % 8)
          if k < 6:
            idx_wait((k + 2) % 8)
            gather_start(gg, (k + 2) % 4, (k + 2) % 8)
          else:
            @pl.when(j < NJ - 1)
            def _():
              idx_wait((k + 2) % 8)
              gather_start(gg, (k + 2) % 4, (k + 2) % 8)
        return carry

      lax.fori_loop(0, NJ, body, 0)
      scatter_wait(2, 6)
      scatter_wait(3, 7)
      plsc.subcore_barrier()
      pltpu.sync_copy(acc.at[pl.ds(rbase, ROWS_PT)], out_hbm.at[gg].at[sid])
      return carry

    lax.fori_loop(0, G // 2, g_body, 0)

  def wrapped(x, src, dst, zeros_blk):
    return seg_sum(x, src, dst, zeros_blk).reshape(G, N, D)

  return wrapped


BLK = 2000
NB = N // BLK


def _layer1_body(x_ref, agg_ref, w1r_ref, w1n_ref, b1_ref, out_ref):
  z = (jnp.dot(x_ref[0], w1r_ref[...], preferred_element_type=jnp.float32)
       + jnp.dot(agg_ref[0], w1n_ref[...], preferred_element_type=jnp.float32)
       + b1_ref[0])
  out_ref[0, 0] = jnp.maximum(z, 0.0)


def _layer1(x_seq, agg1, W1r, W1n, b1):
  return pl.pallas_call(
      _layer1_body,
      grid=(T, NB, 2),
      in_specs=[
          pl.BlockSpec((1, BLK, D), lambda t, nb, h: (t, nb, 0)),
          pl.BlockSpec((1, BLK, D), lambda t, nb, h: (t, nb, 0)),
          pl.BlockSpec((D, H // 2), lambda t, nb, h: (0, h)),
          pl.BlockSpec((D, H // 2), lambda t, nb, h: (0, h)),
          pl.BlockSpec((1, 1, H // 2), lambda t, nb, h: (h, 0, 0)),
      ],
      out_specs=pl.BlockSpec((1, 1, BLK, D), lambda t, nb, h: (h, t, nb, 0)),
      out_shape=jax.ShapeDtypeStruct((2, T, N, D), jnp.float32),
  )(x_seq, agg1, W1r, W1n, b1.reshape(2, 1, H // 2))


def _layer2_body(x0_ref, x1_ref, a0_ref, a1_ref, w2r_ref, w2n_ref, b2_ref,
                 out_ref):
  nb = pl.program_id(1)
  z = (jnp.dot(x0_ref[0, 0], w2r_ref[:D, :], preferred_element_type=jnp.float32)
       + jnp.dot(x1_ref[0, 0], w2r_ref[D:, :], preferred_element_type=jnp.float32)
       + jnp.dot(a0_ref[0], w2n_ref[:D, :], preferred_element_type=jnp.float32)
       + jnp.dot(a1_ref[0], w2n_ref[D:, :], preferred_element_type=jnp.float32)
       + b2_ref[...])
  x3 = jnp.maximum(z, 0.0)
  part = jnp.sum(x3, axis=0, keepdims=True) * (1.0 / N)

  @pl.when(nb == 0)
  def _():
    out_ref[0] = part

  @pl.when(nb != 0)
  def _():
    out_ref[0] = out_ref[0] + part


def _layer2_pool(x2, agg2, W2r, W2n, b2):
  return pl.pallas_call(
      _layer2_body,
      grid=(T, NB),
      in_specs=[
          pl.BlockSpec((1, 1, BLK, D), lambda t, nb: (0, t, nb, 0)),
          pl.BlockSpec((1, 1, BLK, D), lambda t, nb: (1, t, nb, 0)),
          pl.BlockSpec((1, BLK, D), lambda t, nb: (t, nb, 0)),
          pl.BlockSpec((1, BLK, D), lambda t, nb: (t + T, nb, 0)),
          pl.BlockSpec((H, H), lambda t, nb: (0, 0)),
          pl.BlockSpec((H, H), lambda t, nb: (0, 0)),
          pl.BlockSpec((1, H), lambda t, nb: (0, 0)),
      ],
      out_specs=pl.BlockSpec((1, 1, H), lambda t, nb: (t, 0, 0)),
      out_shape=jax.ShapeDtypeStruct((T, 1, H), jnp.float32),
  )(x2, x2, agg2, agg2, W2r, W2n, b2.reshape(1, H)).reshape(T, H)


def _lstm_body(seq_ref, wih_ref, whh_ref, b_ref, wc_ref, bc_ref, out_ref):
  zx = jnp.dot(seq_ref[...], wih_ref[...],
               preferred_element_type=jnp.float32) + b_ref[...]

  def step(t, hc):
    h, c = hc
    onehot = (lax.broadcasted_iota(jnp.int32, (1, T), 1) == t).astype(jnp.float32)
    row = jnp.dot(onehot, zx, preferred_element_type=jnp.float32)
    z = row + jnp.dot(h, whh_ref[...], preferred_element_type=jnp.float32)
    i = jax.nn.sigmoid(z[:, :L])
    f = jax.nn.sigmoid(z[:, L:2 * L])
    g = jnp.tanh(z[:, 2 * L:3 * L])
    o = jax.nn.sigmoid(z[:, 3 * L:])
    c = f * c + i * g
    h = o * jnp.tanh(c)
    return (h, c)

  h0 = jnp.zeros((1, L), jnp.float32)
  c0 = jnp.zeros((1, L), jnp.float32)
  h, _ = lax.fori_loop(0, T, step, (h0, c0))
  out_ref[...] = jnp.dot(h, wc_ref[...],
                         preferred_element_type=jnp.float32) + bc_ref[...]


def _lstm_head(pooled, W_ih, W_hh, bsum, Wc_pad, bc_pad):
  return pl.pallas_call(
      _lstm_body,
      out_shape=jax.ShapeDtypeStruct((1, 128), jnp.float32),
  )(pooled, W_ih, W_hh, bsum, Wc_pad, bc_pad)


def kernel(x_seq, edge_index, W1r, W1n, b1, W2r, W2n, b2,
           W_ih, W_hh, b_ih, b_hh, Wc, bc):
  ei = edge_index.astype(jnp.int32)
  npad = EPAD - E
  src = jnp.concatenate([ei[0], jnp.zeros((npad,), jnp.int32)])
  dst = jnp.concatenate([ei[1], jnp.full((npad,), N, jnp.int32)])
  zeros_blk = jnp.zeros((ROWS_PT, D), jnp.float32)

  agg1 = _make_seg_sum(T)(x_seq, src, dst, zeros_blk)
  x2 = _layer1(x_seq, agg1, W1r, W1n, b1)
  agg2 = _make_seg_sum(2 * T)(x2.reshape(2 * T, N, D), src, dst, zeros_blk)
  pooled = _layer2_pool(x2, agg2, W2r, W2n, b2)

  bsum = (b_ih + b_hh).reshape(1, 4 * L)
  Wc_pad = jnp.pad(Wc, ((0, 0), (0, 128 - C)))
  bc_pad = jnp.pad(bc, (0, 128 - C)).reshape(1, 128)
  logits = _lstm_head(pooled, W_ih, W_hh, bsum, Wc_pad, bc_pad)
  return logits[:, :C]


# speedup vs baseline: 1.3363x; 1.3363x over previous
import functools

import jax
import jax.numpy as jnp
from jax import lax
from jax.experimental import pallas as pl
from jax.experimental.pallas import tpu as pltpu
from jax.experimental.pallas import tpu_sc as plsc

N = 10000
E = 320000
T = 16
D = 128
H = 256
L = 256
C = 5

NTILES = 16
ROWS_PT = N // NTILES
K = 128
NCH = 158
NPAIR = NCH // 2
EPT = NCH * K
EPAD = NTILES * EPT
ACC_N = N + 16


@functools.lru_cache(maxsize=None)
def _make_seg_sum(G):
  mesh = plsc.VectorSubcoreMesh(core_axis_name="c", subcore_axis_name="s")

  @functools.partial(
      pl.kernel,
      out_type=jax.ShapeDtypeStruct((G, NTILES, ROWS_PT, D), jnp.float32),
      mesh=mesh,
      scratch_types=[
          [pltpu.VMEM((K,), jnp.int32) for _ in range(2)],
          [pltpu.VMEM((K,), jnp.int32) for _ in range(2)],
          [pltpu.VMEM((K, D), jnp.float32) for _ in range(2)],
          pltpu.VMEM_SHARED((ACC_N, D), jnp.float32),
          [pltpu.SemaphoreType.DMA for _ in range(2)],
          [pltpu.SemaphoreType.DMA for _ in range(2)],
          pltpu.SemaphoreType.DMA,
      ],
  )
  def seg_sum(x_hbm, src_hbm, dst_hbm, zeros_hbm, out_hbm,
              srcv, dstv, rowsv, acc, isem, gsem, ssem):
    cid = lax.axis_index("c")
    sid = lax.axis_index("s")
    ebase = sid * EPT
    rbase = sid * ROWS_PT

    def idx_start(c, s):
      off = ebase + c * K
      pltpu.async_copy(src_hbm.at[pl.ds(off, K)], srcv[s], isem[s])
      pltpu.async_copy(dst_hbm.at[pl.ds(off, K)], dstv[s], isem[s])

    def idx_wait(s):
      pltpu.make_async_copy(src_hbm.at[pl.ds(0, K)], srcv[s], isem[s]).wait()
      pltpu.make_async_copy(dst_hbm.at[pl.ds(0, K)], dstv[s], isem[s]).wait()

    def gather_start(gg, s):
      pltpu.async_copy(x_hbm.at[gg].at[srcv[s]], rowsv[s], gsem[s])

    def gather_wait(gg, s):
      pltpu.make_async_copy(x_hbm.at[gg].at[srcv[s]], rowsv[s],
                            gsem[s]).wait()

    def scatter_start(s):
      return pltpu.async_copy(rowsv[s], acc.at[dstv[s]], ssem, add=True)

    def g_body(gl, carry):
      gg = 2 * gl + cid
      pltpu.sync_copy(zeros_hbm, acc.at[pl.ds(rbase, ROWS_PT)])
      plsc.subcore_barrier()

      idx_start(0, 0)
      idx_wait(0)
      gather_start(gg, 0)
      idx_start(1, 1)

      def body(j, carry):
        a = 2 * j
        gather_wait(gg, 0)
        sa = scatter_start(0)
        idx_wait(1)
        gather_start(gg, 1)
        sa.wait()

        @pl.when(j < NPAIR - 1)
        def _():
          idx_start(a + 2, 0)

        gather_wait(gg, 1)
        sb = scatter_start(1)

        @pl.when(j < NPAIR - 1)
        def _():
          idx_wait(0)
          gather_start(gg, 0)

        sb.wait()

        @pl.when(j < NPAIR - 1)
        def _():
          idx_start(a + 3, 1)

        return carry

      lax.fori_loop(0, NPAIR, body, 0)
      plsc.subcore_barrier()
      pltpu.sync_copy(acc.at[pl.ds(rbase, ROWS_PT)], out_hbm.at[gg].at[sid])
      return carry

    lax.fori_loop(0, G // 2, g_body, 0)

  def wrapped(x, src, dst, zeros_blk):
    return seg_sum(x, src, dst, zeros_blk).reshape(G, N, D)

  return wrapped


BLK = 2000
NB = N // BLK


def _layer1_body(x_ref, agg_ref, w1r_ref, w1n_ref, b1_ref, out_ref):
  agg = agg_ref[0].astype(jnp.float32)
  z = (jnp.dot(x_ref[0], w1r_ref[...], preferred_element_type=jnp.float32)
       + jnp.dot(agg, w1n_ref[...], preferred_element_type=jnp.float32)
       + b1_ref[0])
  out_ref[0, 0] = jnp.maximum(z, 0.0)


def _layer1(x_seq, agg1, W1r, W1n, b1):
  return pl.pallas_call(
      _layer1_body,
      grid=(T, NB, 2),
      in_specs=[
          pl.BlockSpec((1, BLK, D), lambda t, nb, h: (t, nb, 0)),
          pl.BlockSpec((1, BLK, D), lambda t, nb, h: (t, nb, 0)),
          pl.BlockSpec((D, H // 2), lambda t, nb, h: (0, h)),
          pl.BlockSpec((D, H // 2), lambda t, nb, h: (0, h)),
          pl.BlockSpec((1, 1, H // 2), lambda t, nb, h: (h, 0, 0)),
      ],
      out_specs=pl.BlockSpec((1, 1, BLK, D), lambda t, nb, h: (h, t, nb, 0)),
      out_shape=jax.ShapeDtypeStruct((2, T, N, D), jnp.float32),
  )(x_seq, agg1, W1r, W1n, b1.reshape(2, 1, H // 2))


def _layer2_body(x0_ref, x1_ref, a0_ref, a1_ref, w2r_ref, w2n_ref, b2_ref,
                 out_ref):
  nb = pl.program_id(1)
  x0 = x0_ref[0, 0].astype(jnp.float32)
  x1 = x1_ref[0, 0].astype(jnp.float32)
  a0 = a0_ref[0].astype(jnp.float32)
  a1 = a1_ref[0].astype(jnp.float32)
  z = (jnp.dot(x0, w2r_ref[:D, :], preferred_element_type=jnp.float32)
       + jnp.dot(x1, w2r_ref[D:, :], preferred_element_type=jnp.float32)
       + jnp.dot(a0, w2n_ref[:D, :], preferred_element_type=jnp.float32)
       + jnp.dot(a1, w2n_ref[D:, :], preferred_element_type=jnp.float32)
       + b2_ref[...])
  x3 = jnp.maximum(z, 0.0)
  part = jnp.sum(x3, axis=0, keepdims=True) * (1.0 / N)

  @pl.when(nb == 0)
  def _():
    out_ref[0] = part

  @pl.when(nb != 0)
  def _():
    out_ref[0] = out_ref[0] + part


def _layer2_pool(x2, agg2, W2r, W2n, b2):
  return pl.pallas_call(
      _layer2_body,
      grid=(T, NB),
      in_specs=[
          pl.BlockSpec((1, 1, BLK, D), lambda t, nb: (0, t, nb, 0)),
          pl.BlockSpec((1, 1, BLK, D), lambda t, nb: (1, t, nb, 0)),
          pl.BlockSpec((1, BLK, D), lambda t, nb: (t, nb, 0)),
          pl.BlockSpec((1, BLK, D), lambda t, nb: (t + T, nb, 0)),
          pl.BlockSpec((H, H), lambda t, nb: (0, 0)),
          pl.BlockSpec((H, H), lambda t, nb: (0, 0)),
          pl.BlockSpec((1, H), lambda t, nb: (0, 0)),
      ],
      out_specs=pl.BlockSpec((1, 1, H), lambda t, nb: (t, 0, 0)),
      out_shape=jax.ShapeDtypeStruct((T, 1, H), jnp.float32),
  )(x2, x2, agg2, agg2, W2r, W2n, b2.reshape(1, H)).reshape(T, H)


def _lstm_body(seq_ref, wih_ref, whh_ref, b_ref, wc_ref, bc_ref, out_ref):
  zx = jnp.dot(seq_ref[...], wih_ref[...],
               preferred_element_type=jnp.float32) + b_ref[...]

  def step(t, hc):
    h, c = hc
    onehot = (lax.broadcasted_iota(jnp.int32, (1, T), 1) == t).astype(jnp.float32)
    row = jnp.dot(onehot, zx, preferred_element_type=jnp.float32)
    z = row + jnp.dot(h, whh_ref[...], preferred_element_type=jnp.float32)
    i = jax.nn.sigmoid(z[:, :L])
    f = jax.nn.sigmoid(z[:, L:2 * L])
    g = jnp.tanh(z[:, 2 * L:3 * L])
    o = jax.nn.sigmoid(z[:, 3 * L:])
    c = f * c + i * g
    h = o * jnp.tanh(c)
    return (h, c)

  h0 = jnp.zeros((1, L), jnp.float32)
  c0 = jnp.zeros((1, L), jnp.float32)
  h, _ = lax.fori_loop(0, T, step, (h0, c0))
  out_ref[...] = jnp.dot(h, wc_ref[...],
                         preferred_element_type=jnp.float32) + bc_ref[...]


def _lstm_head(pooled, W_ih, W_hh, bsum, Wc_pad, bc_pad):
  return pl.pallas_call(
      _lstm_body,
      out_shape=jax.ShapeDtypeStruct((1, 128), jnp.float32),
  )(pooled, W_ih, W_hh, bsum, Wc_pad, bc_pad)


def kernel(x_seq, edge_index, W1r, W1n, b1, W2r, W2n, b2,
           W_ih, W_hh, b_ih, b_hh, Wc, bc):
  ei = edge_index.astype(jnp.int32)
  npad = EPAD - E
  src = jnp.concatenate([ei[0], jnp.zeros((npad,), jnp.int32)])
  dst = jnp.concatenate([ei[1], jnp.full((npad,), N, jnp.int32)])
  zeros_blk = jnp.zeros((ROWS_PT, D), jnp.float32)

  agg1 = _make_seg_sum(T)(x_seq, src, dst, zeros_blk)
  x2 = _layer1(x_seq, agg1, W1r, W1n, b1)
  agg2 = _make_seg_sum(2 * T)(x2.reshape(2 * T, N, D), src, dst, zeros_blk)
  pooled = _layer2_pool(x2, agg2, W2r, W2n, b2)

  bsum = (b_ih + b_hh).reshape(1, 4 * L)
  Wc_pad = jnp.pad(Wc, ((0, 0), (0, 128 - C)))
  bc_pad = jnp.pad(bc, (0, 128 - C)).reshape(1, 128)
  logits = _lstm_head(pooled, W_ih, W_hh, bsum, Wc_pad, bc_pad)
  return logits[:, :C]
